# baseline (speedup 1.0000x reference)
import jax
import jax.numpy as jnp
from jax import lax
from jax.experimental import pallas as pl
from jax.experimental.pallas import tpu as pltpu

_LANES = 128
_G = 8
_NBUF = 16


def _body(perm_ref, nxt_ref, nn_ref, nz_ref, mask_ref, x_ref, o_ref,
          mask_v, buf, sems, msem):
    j = pl.program_id(0)
    nn = nn_ref[0]
    nz = nz_ref[0]

    def issue(c):
        blk = nxt_ref[c]
        slot = lax.rem(c, _NBUF)
        pltpu.make_async_copy(
            x_ref.at[:, pl.ds(blk * _LANES, _LANES)],
            buf.at[slot],
            sems.at[slot],
        ).start()

    @pl.when(j == 0)
    def _prime():
        cp = pltpu.make_async_copy(mask_ref, mask_v, msem)
        cp.start()
        cp.wait()
        for i in range(_G):
            @pl.when(i < nn)
            def _(i=i):
                issue(i)

    @pl.when(j < nz)
    def _zero():
        o_ref[...] = jnp.zeros_like(o_ref)

    @pl.when(j >= nz)
    def _needed():
        c = j - nz

        @pl.when(lax.rem(c, _G) == 0)
        def _group_boundary():
            for i in range(_G):
                @pl.when(c + i < nn)
                def _(i=i):
                    slot = lax.rem(c + i, _NBUF)
                    pltpu.make_async_copy(
                        x_ref.at[:, pl.ds(nxt_ref[c + i] * _LANES, _LANES)],
                        buf.at[slot],
                        sems.at[slot],
                    ).wait()
            for i in range(_G):
                @pl.when(c + _G + i < nn)
                def _(i=i):
                    issue(c + _G + i)

        blk = nxt_ref[c]
        slot = lax.rem(c, _NBUF)
        o_ref[...] = buf[slot] * mask_v[pl.ds(blk, 1), :]


def kernel(x, neuron_indices, K):
    batch, d_sae = x.shape
    nb = d_sae // _LANES

    in_first_K = jnp.arange(d_sae, dtype=jnp.int32) < K
    mask = (
        jnp.zeros((d_sae,), jnp.bool_)
        .at[neuron_indices]
        .max(in_first_K)
        .astype(jnp.float32)
    )
    mask_blocks = mask.reshape(nb, _LANES)
    needed = (mask_blocks.max(axis=1) > 0).astype(jnp.int32)
    incl = jnp.cumsum(needed, dtype=jnp.int32)
    cnt = incl - needed
    nn = incl[-1:]
    nz = jnp.int32(nb) - nn
    nxt = (
        jnp.zeros((nb,), jnp.int32)
        .at[jnp.where(needed == 1, cnt, nb)]
        .set(jnp.arange(nb, dtype=jnp.int32), mode="drop")
    )
    perm = jnp.argsort(needed, stable=True).astype(jnp.int32)

    grid_spec = pltpu.PrefetchScalarGridSpec(
        num_scalar_prefetch=4,
        grid=(nb,),
        in_specs=[
            pl.BlockSpec(memory_space=pl.ANY),
            pl.BlockSpec(memory_space=pl.ANY),
        ],
        out_specs=pl.BlockSpec(
            (batch, _LANES), lambda j, perm, nxt, nn, nz: (0, perm[j])
        ),
        scratch_shapes=[
            pltpu.VMEM((nb, _LANES), jnp.float32),
            pltpu.VMEM((_NBUF, batch, _LANES), jnp.float32),
            pltpu.SemaphoreType.DMA((_NBUF,)),
            pltpu.SemaphoreType.DMA,
        ],
    )

    return pl.pallas_call(
        _body,
        grid_spec=grid_spec,
        out_shape=jax.ShapeDtypeStruct((batch, d_sae), x.dtype),
    )(perm, nxt, nn, nz, mask_blocks, x)
